# Initial kernel scaffold; baseline (speedup 1.0000x reference)
import functools

import jax
import jax.numpy as jnp
from jax import lax
from jax.experimental import pallas as pl
from jax.experimental.pallas import tpu as pltpu
from jax.experimental.pallas import tpu_sc as plsc

D = 32
B = 16384 * 50
NC, NS = 2, 16
NW = NC * NS
GROUP = 128
K = 10
CHUNK = GROUP * K
GROUPS_PER_W = B // (NW * GROUP)
CHUNKS_PER_W = GROUPS_PER_W // K

_mesh = plsc.VectorSubcoreMesh(core_axis_name="c", subcore_axis_name="s")


@functools.partial(
    pl.kernel,
    out_type=jax.ShapeDtypeStruct((B, D), jnp.float32),
    mesh=_mesh,
    scratch_types=[
        pltpu.VMEM((K, GROUP), jnp.int32),
        pltpu.VMEM((CHUNK, D), jnp.float32),
        pltpu.SemaphoreType.DMA,
    ],
)
def _gather_kernel(idx_hbm, table_hbm, out_hbm, idx_v, rows_v, sem):
    wid = lax.axis_index("s") * NC + lax.axis_index("c")
    base_group = wid * GROUPS_PER_W

    def chunk_body(chunk, carry):
        g0 = base_group + chunk * K
        pltpu.sync_copy(idx_hbm.at[pl.ds(g0, K)], idx_v)
        copies = [
            pltpu.async_copy(
                table_hbm.at[idx_v.at[j]],
                rows_v.at[pl.ds(j * GROUP, GROUP)],
                sem,
            )
            for j in range(K)
        ]
        for cp in copies:
            cp.wait()
        pltpu.sync_copy(rows_v, out_hbm.at[pl.ds(g0 * GROUP, CHUNK)])
        return carry

    lax.fori_loop(0, CHUNKS_PER_W, chunk_body, 0)


def kernel(x, W):
    idx = x.reshape(B // GROUP, GROUP).astype(jnp.int32)
    out = _gather_kernel(idx, W)
    return out.reshape(x.shape[0], x.shape[1], D)


# speedup vs baseline: 1.0945x; 1.0945x over previous
import functools

import jax
import jax.numpy as jnp
from jax import lax
from jax.experimental import pallas as pl
from jax.experimental.pallas import tpu as pltpu
from jax.experimental.pallas import tpu_sc as plsc

D = 32
B = 16384 * 50
NC, NS = 2, 16
NW = NC * NS
GROUP = 128
K = 8
CHUNK = GROUP * K
GROUPS_PER_W = B // (NW * GROUP)
CHUNKS_PER_W = GROUPS_PER_W // K

_mesh = plsc.VectorSubcoreMesh(core_axis_name="c", subcore_axis_name="s")


@functools.partial(
    pl.kernel,
    out_type=jax.ShapeDtypeStruct((B, D), jnp.float32),
    mesh=_mesh,
    scratch_types=[
        pltpu.VMEM((K, GROUP), jnp.int32),
        pltpu.VMEM((CHUNK, D), jnp.float32),
        pltpu.SemaphoreType.DMA,
    ],
    compiler_params=pltpu.CompilerParams(use_tc_tiling_on_sc=False),
)
def _gather_kernel(idx_hbm, table_hbm, out_hbm, idx_v, rows_v, sem):
    wid = lax.axis_index("s") * NC + lax.axis_index("c")
    base_group = wid * GROUPS_PER_W

    def chunk_body(chunk, carry):
        g0 = base_group + chunk * K
        pltpu.sync_copy(idx_hbm.at[pl.ds(g0, K)], idx_v)
        copies = [
            pltpu.async_copy(
                table_hbm.at[idx_v.at[j]],
                rows_v.at[pl.ds(j * GROUP, GROUP)],
                sem,
            )
            for j in range(K)
        ]
        for cp in copies:
            cp.wait()
        pltpu.sync_copy(rows_v, out_hbm.at[pl.ds(g0 * GROUP, CHUNK)])
        return carry

    lax.fori_loop(0, CHUNKS_PER_W, chunk_body, 0)


def kernel(x, W):
    idx = x.reshape(B // GROUP, GROUP).astype(jnp.int32)
    out = _gather_kernel(idx, W)
    return out.reshape(x.shape[0], x.shape[1], D)


# speedup vs baseline: 1.1110x; 1.0151x over previous
import functools

import jax
import jax.numpy as jnp
from jax import lax
from jax.experimental import pallas as pl
from jax.experimental.pallas import tpu as pltpu
from jax.experimental.pallas import tpu_sc as plsc

D = 32
B = 16384 * 50
NC, NS = 2, 16
NW = NC * NS
GROUP = 128
K = 8
CHUNK = GROUP * K
GROUPS_PER_W = B // (NW * GROUP)
NCH = GROUPS_PER_W // K

_mesh = plsc.VectorSubcoreMesh(core_axis_name="c", subcore_axis_name="s")


@functools.partial(
    pl.kernel,
    out_type=jax.ShapeDtypeStruct((B, D), jnp.float32),
    mesh=_mesh,
    scratch_types=[
        pltpu.VMEM((GROUPS_PER_W, GROUP), jnp.int32),
        pltpu.VMEM((CHUNK, D), jnp.float32),
        pltpu.VMEM((CHUNK, D), jnp.float32),
        pltpu.SemaphoreType.DMA,
        pltpu.SemaphoreType.DMA,
        pltpu.SemaphoreType.DMA,
        pltpu.SemaphoreType.DMA,
    ],
    compiler_params=pltpu.CompilerParams(use_tc_tiling_on_sc=False),
)
def _gather_kernel(idx_hbm, table_hbm, out_hbm, idx_all, rows0, rows1,
                   sg0, sg1, ss0, ss1):
    wid = lax.axis_index("s") * NC + lax.axis_index("c")
    base_group = wid * GROUPS_PER_W
    rows_v = (rows0, rows1)
    sg = (sg0, sg1)
    ss = (ss0, ss1)

    def fire(t, b):
        for j in range(K):
            pltpu.async_copy(
                table_hbm.at[idx_all.at[t * K + j]],
                rows_v[b].at[pl.ds(j * GROUP, GROUP)],
                sg[b],
            )

    def drain_gather(b):
        pltpu.make_async_copy(
            table_hbm.at[pl.ds(0, CHUNK)], rows_v[b], sg[b]).wait()

    def store(t, b):
        g0 = base_group + t * K
        pltpu.async_copy(
            rows_v[b], out_hbm.at[pl.ds(g0 * GROUP, CHUNK)], ss[b])

    def drain_store(b):
        pltpu.make_async_copy(
            rows_v[b], out_hbm.at[pl.ds(0, CHUNK)], ss[b]).wait()

    pltpu.sync_copy(idx_hbm.at[pl.ds(base_group, GROUPS_PER_W)], idx_all)
    fire(0, 0)

    def body(i, carry):
        t = 2 * i
        drain_gather(0)

        @pl.when(i >= 1)
        def _():
            drain_store(1)

        fire(t + 1, 1)
        store(t, 0)
        drain_gather(1)
        drain_store(0)
        fire(t + 2, 0)
        store(t + 1, 1)
        return carry

    lax.fori_loop(0, (NCH - 1) // 2, body, 0)

    drain_gather(0)
    drain_store(1)
    store(NCH - 1, 0)
    drain_store(0)


def kernel(x, W):
    idx = x.reshape(B // GROUP, GROUP).astype(jnp.int32)
    out = _gather_kernel(idx, W)
    return out.reshape(x.shape[0], x.shape[1], D)


# speedup vs baseline: 1.7628x; 1.5866x over previous
import functools

import jax
import jax.numpy as jnp
from jax import lax
from jax.experimental import pallas as pl
from jax.experimental.pallas import tpu as pltpu
from jax.experimental.pallas import tpu_sc as plsc

BATCH = 16384
HIST = 50
D = 32
NC, NS = 2, 16
NW = NC * NS
ROWS_PER_W = BATCH // NW
R = 8
NCH = ROWS_PER_W // R

_mesh = plsc.VectorSubcoreMesh(core_axis_name="c", subcore_axis_name="s")


@functools.partial(
    pl.kernel,
    out_type=jax.ShapeDtypeStruct((BATCH, HIST, D), jnp.float32),
    mesh=_mesh,
    scratch_types=[
        pltpu.VMEM((ROWS_PER_W, HIST), jnp.int32),
        pltpu.VMEM((R, HIST, D), jnp.float32),
        pltpu.VMEM((R, HIST, D), jnp.float32),
        pltpu.SemaphoreType.DMA,
        pltpu.SemaphoreType.DMA,
        pltpu.SemaphoreType.DMA,
        pltpu.SemaphoreType.DMA,
    ],
    compiler_params=pltpu.CompilerParams(use_tc_tiling_on_sc=False),
)
def _gather_kernel(x_hbm, table_hbm, out_hbm, idx_all, rows0, rows1,
                   sg0, sg1, ss0, ss1):
    wid = lax.axis_index("s") * NC + lax.axis_index("c")
    base_row = wid * ROWS_PER_W
    rows_v = (rows0, rows1)
    sg = (sg0, sg1)
    ss = (ss0, ss1)

    def fire(t, b):
        for i in range(R):
            pltpu.async_copy(
                table_hbm.at[idx_all.at[t * R + i]],
                rows_v[b].at[i],
                sg[b],
            )

    def drain_gather(b):
        for i in range(R):
            pltpu.make_async_copy(
                table_hbm.at[pl.ds(0, HIST)], rows_v[b].at[i], sg[b]).wait()

    def store(t, b):
        pltpu.async_copy(
            rows_v[b], out_hbm.at[pl.ds(base_row + t * R, R)], ss[b])

    def drain_store(b):
        pltpu.make_async_copy(
            rows_v[b], out_hbm.at[pl.ds(0, R)], ss[b]).wait()

    pltpu.sync_copy(x_hbm.at[pl.ds(base_row, ROWS_PER_W)], idx_all)
    fire(0, 0)

    def body(i, carry):
        t = 2 * i
        drain_gather(0)

        @pl.when(i >= 1)
        def _():
            drain_store(1)

        fire(t + 1, 1)
        store(t, 0)
        drain_gather(1)
        drain_store(0)

        @pl.when(t + 2 < NCH)
        def _():
            fire(t + 2, 0)

        store(t + 1, 1)
        return carry

    lax.fori_loop(0, NCH // 2, body, 0)

    drain_store(1)


def kernel(x, W):
    return _gather_kernel(x.astype(jnp.int32), W)


# speedup vs baseline: 1.9082x; 1.0825x over previous
import functools

import jax
import jax.numpy as jnp
from jax import lax
from jax.experimental import pallas as pl
from jax.experimental.pallas import tpu as pltpu
from jax.experimental.pallas import tpu_sc as plsc

BATCH = 16384
HIST = 50
D = 32
B = BATCH * HIST
NC, NS = 2, 16
NW = NC * NS
COLS_PER_W = BATCH // NW
LOOK_PER_W = HIST * COLS_PER_W
CHUNK = COLS_PER_W
GR = CHUNK // D
NCH = HIST

_mesh = plsc.VectorSubcoreMesh(core_axis_name="c", subcore_axis_name="s")


@functools.partial(
    pl.kernel,
    out_type=jax.ShapeDtypeStruct((B, D), jnp.float32),
    mesh=_mesh,
    scratch_types=[
        pltpu.VMEM((LOOK_PER_W,), jnp.int32),
        pltpu.VMEM((CHUNK, D), jnp.float32),
        pltpu.VMEM((CHUNK, D), jnp.float32),
        pltpu.SemaphoreType.DMA,
        pltpu.SemaphoreType.DMA,
        pltpu.SemaphoreType.DMA,
        pltpu.SemaphoreType.DMA,
    ],
    compiler_params=pltpu.CompilerParams(use_tc_tiling_on_sc=False),
)
def _gather_kernel(xt_hbm, table_hbm, out_hbm, idx_v, rows0, rows1,
                   sg0, sg1, ss0, ss1):
    wid = lax.axis_index("s") * NC + lax.axis_index("c")
    col0 = wid * COLS_PER_W
    rows_v = (rows0, rows1)
    sg = (sg0, sg1)
    ss = (ss0, ss1)

    def fire(t, b):
        for j in range(GR):
            pltpu.async_copy(
                table_hbm.at[idx_v.at[pl.ds(t * CHUNK + j * D, D)]],
                rows_v[b].at[pl.ds(j * D, D)],
                sg[b],
            )

    def drain_gather(b):
        pltpu.make_async_copy(
            table_hbm.at[pl.ds(0, CHUNK)], rows_v[b], sg[b]).wait()

    def store(t, b):
        pltpu.async_copy(
            rows_v[b],
            out_hbm.at[pl.ds(t * BATCH + col0, CHUNK)],
            ss[b],
        )

    def drain_store(b):
        pltpu.make_async_copy(
            rows_v[b], out_hbm.at[pl.ds(0, CHUNK)], ss[b]).wait()

    stage = [
        pltpu.async_copy(
            xt_hbm.at[h, pl.ds(col0, COLS_PER_W)],
            idx_v.at[pl.ds(h * COLS_PER_W, COLS_PER_W)],
            sg0,
        )
        for h in range(HIST)
    ]
    for cp in stage:
        cp.wait()
    fire(0, 0)

    def body(i, carry):
        t = 2 * i
        drain_gather(0)

        @pl.when(i >= 1)
        def _():
            drain_store(1)

        fire(t + 1, 1)
        store(t, 0)
        drain_gather(1)
        drain_store(0)

        @pl.when(t + 2 < NCH)
        def _():
            fire(t + 2, 0)

        store(t + 1, 1)
        return carry

    lax.fori_loop(0, NCH // 2, body, 0)

    drain_store(1)


def kernel(x, W):
    out = _gather_kernel(jnp.transpose(x).astype(jnp.int32), W)
    return jnp.transpose(out.reshape(HIST, BATCH, D), (1, 0, 2))
